# baseline (speedup 1.0000x reference)
from functools import partial

import jax
import jax.numpy as jnp
from jax.experimental import pallas as pl
from jax.experimental.pallas import tpu as pltpu


def _ca_fused_kernel(x_ref, w1t_ref, b1_ref, w2t_ref, b2_ref, o_ref, *,
                     inv_hw, w):
    x = x_ref[...]
    if w % 128 != 0:
        lane = jax.lax.broadcasted_iota(jnp.int32, x.shape, dimension=3)
        x = jnp.where(lane < w, x, 0.0)
    y = jnp.sum(x, axis=(-2, -1), dtype=jnp.float32) * inv_hw

    t1 = jnp.dot(y, w1t_ref[...], preferred_element_type=jnp.float32)
    t1 = jnp.maximum(t1 + b1_ref[...], 0.0)
    t2 = jnp.dot(t1, w2t_ref[...], preferred_element_type=jnp.float32)
    scale = jax.nn.sigmoid(t2 + b2_ref[...]).astype(x_ref.dtype)

    o_ref[...] = (x_ref[...] * scale[:, :, None, None]).astype(o_ref.dtype)


def kernel(x, w1, b1, w2, b2):
    B, C, H, W = x.shape
    Cr = w1.shape[0]
    inv_hw = float(1.0 / (H * W))

    w1t = jnp.transpose(w1)
    w2t = jnp.transpose(w2)
    b1r = b1.reshape(1, Cr)
    b2r = b2.reshape(1, C)

    out = pl.pallas_call(
        partial(_ca_fused_kernel, inv_hw=inv_hw, w=W),
        out_shape=jax.ShapeDtypeStruct((B, C, H, W), x.dtype),
        grid=(B,),
        in_specs=[
            pl.BlockSpec((1, C, H, W), lambda b: (b, 0, 0, 0)),
            pl.BlockSpec((C, Cr), lambda b: (0, 0)),
            pl.BlockSpec((1, Cr), lambda b: (0, 0)),
            pl.BlockSpec((Cr, C), lambda b: (0, 0)),
            pl.BlockSpec((1, C), lambda b: (0, 0)),
        ],
        out_specs=pl.BlockSpec((1, C, H, W), lambda b: (b, 0, 0, 0)),
        compiler_params=pltpu.CompilerParams(
            dimension_semantics=("parallel",),
            vmem_limit_bytes=48 * 1024 * 1024,
        ),
    )(x, w1t, b1r, w2t, b2r)

    return out
